# Initial kernel scaffold; baseline (speedup 1.0000x reference)
import jax
import jax.numpy as jnp
from jax.experimental import pallas as pl


def kernel(x, edge_index, edge_label_index, W_l1, b_l1, W_r1, W_l2, b_l2, W_r2, W1, b1, W2, b2):
    raise NotImplementedError("write your pallas kernel here")


# speedup vs baseline: 1.0479x; 1.0479x over previous
import functools

import jax
import jax.numpy as jnp
from jax import lax
from jax.experimental import pallas as pl
from jax.experimental.pallas import tpu as pltpu

N = 10000
E = 320000
EQ = 320000
D = 128
H = 128

ROWS_BLK = 1000


def _combine_body(agg_ref, cnt_ref, x_ref, wl_ref, bl_ref, wr_ref, o_ref, *, relu):
    cnt = jnp.maximum(cnt_ref[...], 1.0)
    agg = agg_ref[...] / cnt
    acc = jnp.dot(agg, wl_ref[...].T, preferred_element_type=jnp.float32)
    acc += jnp.dot(x_ref[...], wr_ref[...].T, preferred_element_type=jnp.float32)
    acc += bl_ref[...]
    if relu:
        acc = jnp.maximum(acc, 0.0)
    o_ref[...] = acc


def _combine(aggsum, cnt2d, x, Wl, bl, Wr, relu):
    grid = (N // ROWS_BLK,)
    return pl.pallas_call(
        functools.partial(_combine_body, relu=relu),
        grid=grid,
        in_specs=[
            pl.BlockSpec((ROWS_BLK, H), lambda i: (i, 0)),
            pl.BlockSpec((ROWS_BLK, 1), lambda i: (i, 0)),
            pl.BlockSpec((ROWS_BLK, D), lambda i: (i, 0)),
            pl.BlockSpec((H, D), lambda i: (0, 0)),
            pl.BlockSpec((1, H), lambda i: (0, 0)),
            pl.BlockSpec((H, D), lambda i: (0, 0)),
        ],
        out_specs=pl.BlockSpec((ROWS_BLK, H), lambda i: (i, 0)),
        out_shape=jax.ShapeDtypeStruct((N, H), jnp.float32),
    )(aggsum, cnt2d, x, Wl, bl, Wr)


def _ab_body(z_ref, w1a_ref, w1b_ref, b1_ref, a_ref, b_ref):
    z = z_ref[...]
    a_ref[...] = (
        jnp.dot(z, w1a_ref[...].T, preferred_element_type=jnp.float32) + b1_ref[...]
    )
    b_ref[...] = jnp.dot(z, w1b_ref[...].T, preferred_element_type=jnp.float32)


def _ab(z, W1, b1):
    W1a = W1[:, :H]
    W1b = W1[:, H:]
    grid = (N // ROWS_BLK,)
    return pl.pallas_call(
        _ab_body,
        grid=grid,
        in_specs=[
            pl.BlockSpec((ROWS_BLK, H), lambda i: (i, 0)),
            pl.BlockSpec((H, H), lambda i: (0, 0)),
            pl.BlockSpec((H, H), lambda i: (0, 0)),
            pl.BlockSpec((1, H), lambda i: (0, 0)),
        ],
        out_specs=[
            pl.BlockSpec((ROWS_BLK, H), lambda i: (i, 0)),
            pl.BlockSpec((ROWS_BLK, H), lambda i: (i, 0)),
        ],
        out_shape=[
            jax.ShapeDtypeStruct((N, H), jnp.float32),
            jax.ShapeDtypeStruct((N, H), jnp.float32),
        ],
    )(z, W1a, W1b, b1.reshape(1, H))


def _aggregate(x, src, dst):
    msg = x[src]
    agg = jax.ops.segment_sum(msg, dst, num_segments=N)
    return agg


def _counts(dst):
    return jax.ops.segment_sum(jnp.ones((E,), jnp.float32), dst, num_segments=N)


def kernel(x, edge_index, edge_label_index, W_l1, b_l1, W_r1, W_l2, b_l2, W_r2, W1, b1, W2, b2):
    src = edge_index[0]
    dst = edge_index[1]
    cnt2d = _counts(dst).reshape(N, 1)

    agg1 = _aggregate(x, src, dst)
    z1 = _combine(agg1, cnt2d, x, W_l1, b_l1.reshape(1, H), W_r1, relu=True)

    agg2 = _aggregate(z1, src, dst)
    z2 = _combine(agg2, cnt2d, z1, W_l2, b_l2.reshape(1, H), W_r2, relu=False)

    A, B = _ab(z2, W1, b1)

    s = edge_label_index[0]
    d = edge_label_index[1]
    h = jnp.maximum(A[s] + B[d], 0.0)
    out = h @ W2[0] + b2[0]
    return out.reshape(-1)


# speedup vs baseline: 5.0016x; 4.7731x over previous
import functools

import jax
import jax.numpy as jnp
from jax import lax
from jax.experimental import pallas as pl
from jax.experimental.pallas import tpu as pltpu
from jax.experimental.pallas import tpu_sc as plsc

N = 10000
E = 320000
EQ = 320000
D = 128
H = 128

ROWS_BLK = 1000

NC = 2
NS = 16
NW = NC * NS
CHUNK = 125
E_PER_W = E // NW
NCHUNK = E_PER_W // CHUNK
NP = 10240
IBLK = 8
CB = 2000
ROWS_PER_SUB = NP // NS
ZBLK = 128


def _combine_body(aggp_ref, cnt_ref, x_ref, wl_ref, bl_ref, wr_ref, o_ref, *, relu):
    cnt = jnp.maximum(cnt_ref[...], 1.0)
    agg = (aggp_ref[0] + aggp_ref[1]) / cnt
    acc = jnp.dot(agg, wl_ref[...].T, preferred_element_type=jnp.float32)
    acc += jnp.dot(x_ref[...], wr_ref[...].T, preferred_element_type=jnp.float32)
    acc += bl_ref[...]
    if relu:
        acc = jnp.maximum(acc, 0.0)
    o_ref[...] = acc


def _combine(aggp, cnt2d, x, Wl, bl, Wr, relu):
    grid = (N // ROWS_BLK,)
    return pl.pallas_call(
        functools.partial(_combine_body, relu=relu),
        grid=grid,
        in_specs=[
            pl.BlockSpec((NC, ROWS_BLK, H), lambda i: (0, i, 0)),
            pl.BlockSpec((ROWS_BLK, 1), lambda i: (i, 0)),
            pl.BlockSpec((ROWS_BLK, D), lambda i: (i, 0)),
            pl.BlockSpec((H, D), lambda i: (0, 0)),
            pl.BlockSpec((1, H), lambda i: (0, 0)),
            pl.BlockSpec((H, D), lambda i: (0, 0)),
        ],
        out_specs=pl.BlockSpec((ROWS_BLK, H), lambda i: (i, 0)),
        out_shape=jax.ShapeDtypeStruct((N, H), jnp.float32),
    )(aggp, cnt2d, x, Wl, bl, Wr)


def _ab_body(z_ref, w1a_ref, w1b_ref, b1_ref, a_ref, b_ref):
    z = z_ref[...]
    a_ref[...] = (
        jnp.dot(z, w1a_ref[...].T, preferred_element_type=jnp.float32) + b1_ref[...]
    )
    b_ref[...] = jnp.dot(z, w1b_ref[...].T, preferred_element_type=jnp.float32)


def _ab(z, W1, b1):
    W1a = W1[:, :H]
    W1b = W1[:, H:]
    grid = (N // ROWS_BLK,)
    return pl.pallas_call(
        _ab_body,
        grid=grid,
        in_specs=[
            pl.BlockSpec((ROWS_BLK, H), lambda i: (i, 0)),
            pl.BlockSpec((H, H), lambda i: (0, 0)),
            pl.BlockSpec((H, H), lambda i: (0, 0)),
            pl.BlockSpec((1, H), lambda i: (0, 0)),
        ],
        out_specs=[
            pl.BlockSpec((ROWS_BLK, H), lambda i: (i, 0)),
            pl.BlockSpec((ROWS_BLK, H), lambda i: (i, 0)),
        ],
        out_shape=[
            jax.ShapeDtypeStruct((N, H), jnp.float32),
            jax.ShapeDtypeStruct((N, H), jnp.float32),
        ],
    )(z, W1a, W1b, b1.reshape(1, H))


def _sc_agg_body(table, src3, dst3, zrow, agg_out,
                 src_v, dst_v, rows_v, agg_s, sem):
    cid = lax.axis_index("c")
    sid = lax.axis_index("s")
    wid = sid * NC + cid

    rbase = sid * ROWS_PER_SUB
    for k in range(ROWS_PER_SUB // ZBLK):
        pltpu.sync_copy(zrow, agg_s.at[pl.ds(rbase + k * ZBLK, ZBLK)])
    plsc.subcore_barrier()

    def super_body(g, _):
        pltpu.sync_copy(src3.at[wid, pl.ds(g * IBLK, IBLK)], src_v)
        pltpu.sync_copy(dst3.at[wid, pl.ds(g * IBLK, IBLK)], dst_v)

        def chunk_body(c, _):
            pltpu.async_copy(table.at[src_v.at[c]], rows_v, sem).wait()
            pltpu.sync_copy(rows_v, agg_s.at[dst_v.at[c]], add=True)
            return 0

        lax.fori_loop(0, IBLK, chunk_body, 0)
        return 0

    lax.fori_loop(0, NCHUNK // IBLK, super_body, 0)
    plsc.subcore_barrier()

    for k in range(ROWS_PER_SUB // ZBLK):
        r = rbase + k * ZBLK
        pltpu.sync_copy(agg_s.at[pl.ds(r, ZBLK)], agg_out.at[cid, pl.ds(r, ZBLK)])


def _sc_aggregate(table, src3, dst3, zrow):
    mesh = plsc.VectorSubcoreMesh(core_axis_name="c", subcore_axis_name="s",
                                  num_cores=NC, num_subcores=NS)
    f = pl.kernel(
        _sc_agg_body,
        out_type=jax.ShapeDtypeStruct((NC, NP, D), jnp.float32),
        mesh=mesh,
        scratch_types=[
            pltpu.VMEM((IBLK, CHUNK), jnp.int32),
            pltpu.VMEM((IBLK, CHUNK), jnp.int32),
            pltpu.VMEM((CHUNK, D), jnp.float32),
            pltpu.VMEM_SHARED((NP, D), jnp.float32),
            pltpu.SemaphoreType.DMA,
        ],
    )
    return f(table, src3, dst3, zrow)


DCHUNK = 128
NDC = (EQ // NW) // DCHUNK
DTAIL = EQ // NW - NDC * DCHUNK


def _sc_decode_body(Ah, Bh, s2, d2, wvh, out_h,
                    s_v, d_v, ar0, br0, ar1, br1, art, brt, out_v, wv,
                    semA0, semB0, semA1, semB1):
    cid = lax.axis_index("c")
    sid = lax.axis_index("s")
    wid = sid * NC + cid

    pltpu.sync_copy(wvh, wv)
    pltpu.sync_copy(s2.at[wid, 0], s_v)
    pltpu.sync_copy(d2.at[wid, 0], d_v)

    w2v = tuple(wv[pl.ds(16 * j, 16)] for j in range(D // 16))
    b2v = wv[pl.ds(D, 16)]
    lanes = lax.iota(jnp.int32, 16)

    def fire(c, n, ar, br, semA, semB):
        pltpu.async_copy(Ah.at[s_v.at[pl.ds(c * DCHUNK, n)]], ar, semA)
        pltpu.async_copy(Bh.at[d_v.at[pl.ds(c * DCHUNK, n)]], br, semB)

    def wait(n, ar, br, semA, semB):
        pltpu.make_async_copy(Ah.at[pl.ds(0, n)], ar, semA).wait()
        pltpu.make_async_copy(Bh.at[pl.ds(0, n)], br, semB).wait()

    def compute(c, ngrp, ar, br):
        def group_body(g2, _):
            def edge_body(k, res):
                e = g2 * 16 + k
                acc = b2v
                for j in range(D // 16):
                    a = ar[e, pl.ds(16 * j, 16)]
                    b = br[e, pl.ds(16 * j, 16)]
                    acc = acc + jnp.maximum(a + b, 0.0) * w2v[j]
                for sh in (8, 4, 2, 1):
                    acc = acc + acc[lanes ^ sh]
                return jnp.where(lanes == k, acc, res)

            res = lax.fori_loop(0, 16, edge_body, jnp.zeros((16,), jnp.float32))
            out_v[pl.ds(c * DCHUNK + g2 * 16, 16)] = res
            return 0

        lax.fori_loop(0, ngrp, group_body, 0)

    fire(0, DCHUNK, ar0, br0, semA0, semB0)

    def outer(g, _):
        c0 = 2 * g
        fire(c0 + 1, DCHUNK, ar1, br1, semA1, semB1)
        wait(DCHUNK, ar0, br0, semA0, semB0)
        compute(c0, DCHUNK // 16, ar0, br0)
        fire(jnp.minimum(c0 + 2, NDC - 1), DCHUNK, ar0, br0, semA0, semB0)
        wait(DCHUNK, ar1, br1, semA1, semB1)
        compute(c0 + 1, DCHUNK // 16, ar1, br1)
        return 0

    lax.fori_loop(0, NDC // 2, outer, 0)
    wait(DCHUNK, ar0, br0, semA0, semB0)

    fire(NDC, DTAIL, art, brt, semA0, semB0)
    wait(DTAIL, art, brt, semA0, semB0)
    compute(NDC, DTAIL // 16, art, brt)

    pltpu.sync_copy(out_v, out_h.at[wid, 0])


def _sc_decode(A, B, s2, d2, wv):
    mesh = plsc.VectorSubcoreMesh(core_axis_name="c", subcore_axis_name="s",
                                  num_cores=NC, num_subcores=NS)
    f = pl.kernel(
        _sc_decode_body,
        out_type=jax.ShapeDtypeStruct((NW, 1, EQ // NW), jnp.float32),
        mesh=mesh,
        scratch_types=[
            pltpu.VMEM((EQ // NW,), jnp.int32),
            pltpu.VMEM((EQ // NW,), jnp.int32),
            pltpu.VMEM((DCHUNK, D), jnp.float32),
            pltpu.VMEM((DCHUNK, D), jnp.float32),
            pltpu.VMEM((DCHUNK, D), jnp.float32),
            pltpu.VMEM((DCHUNK, D), jnp.float32),
            pltpu.VMEM((DTAIL, D), jnp.float32),
            pltpu.VMEM((DTAIL, D), jnp.float32),
            pltpu.VMEM((EQ // NW,), jnp.float32),
            pltpu.VMEM((D + 16,), jnp.float32),
            pltpu.SemaphoreType.DMA,
            pltpu.SemaphoreType.DMA,
            pltpu.SemaphoreType.DMA,
            pltpu.SemaphoreType.DMA,
        ],
    )
    return f(A, B, s2, d2, wv)


def kernel(x, edge_index, edge_label_index, W_l1, b_l1, W_r1, W_l2, b_l2, W_r2, W1, b1, W2, b2):
    src3 = edge_index[0].reshape(NW, NCHUNK, CHUNK)
    dst3 = edge_index[1].reshape(NW, NCHUNK, CHUNK)
    zrow = jnp.zeros((ZBLK, D), jnp.float32)
    cnt2d = jax.ops.segment_sum(jnp.ones((E,), jnp.float32), edge_index[1],
                                num_segments=N).reshape(N, 1)

    aggp1 = _sc_aggregate(x, src3, dst3, zrow)[:, :N]
    z1 = _combine(aggp1, cnt2d, x, W_l1, b_l1.reshape(1, H), W_r1, relu=True)

    aggp2 = _sc_aggregate(z1, src3, dst3, zrow)[:, :N]
    z2 = _combine(aggp2, cnt2d, z1, W_l2, b_l2.reshape(1, H), W_r2, relu=False)

    A, B = _ab(z2, W1, b1)

    s2 = edge_label_index[0].reshape(NW, 1, EQ // NW)
    d2 = edge_label_index[1].reshape(NW, 1, EQ // NW)
    wv = jnp.concatenate([W2[0], jnp.full((16,), b2[0] / 16.0, jnp.float32)])
    out = _sc_decode(A, B, s2, d2, wv)
    return out.reshape(-1)
